# baseline (speedup 1.0000x reference)
import functools

import jax
import jax.numpy as jnp
from jax import lax
from jax.experimental import pallas as pl
from jax.experimental.pallas import tpu as pltpu
from jax.experimental.pallas import tpu_sc as plsc

F32 = jnp.float32
I32 = jnp.int32

N = 10000
NP = 10016
NA = 10240
BATCH = 8
C = 128
E = 160000
EP = 163840
PAD_E = EP - E
CHUNK = 128
CHUNKS_PER_TILE = EP // 16 // CHUNK
HIST_PER_TILE = EP // 32
GROWS = BATCH * NP

_mesh = plsc.VectorSubcoreMesh(
    core_axis_name="c", subcore_axis_name="s", num_cores=2, num_subcores=16)


HCHUNKS = EP // CHUNK // 32


@functools.partial(
    pl.kernel,
    out_type=jax.ShapeDtypeStruct((2, NA, C), F32),
    mesh=_mesh,
    compiler_params=pltpu.CompilerParams(needs_layout_passes=False),
    scratch_types=[
        pltpu.VMEM((HCHUNKS, CHUNK), I32),
        pltpu.VMEM((CHUNK, C), F32),
        pltpu.VMEM((16, C), F32),
        pltpu.VMEM_SHARED((NA, C), F32),
    ],
)
def _sc_degree(src_hbm, out_hbm, src_v, ones_v, zero_v, acc_sh):
    c = lax.axis_index("c")
    s = lax.axis_index("s")
    wid = c * 16 + s
    pltpu.sync_copy(src_hbm.at[pl.ds(wid * HCHUNKS, HCHUNKS)], src_v)
    one16 = jnp.ones((16,), F32)
    zero16 = jnp.zeros((16,), F32)
    for i in range(CHUNK):
        for u in range(8):
            ones_v[i, pl.ds(u * 16, 16)] = one16
    for i in range(16):
        for u in range(8):
            zero_v[i, pl.ds(u * 16, 16)] = zero16
    for k in range(40):
        pltpu.sync_copy(zero_v, acc_sh.at[pl.ds(s * 640 + k * 16, 16)])
    plsc.subcore_barrier()

    def ch(j, carry):
        pltpu.sync_copy(ones_v, acc_sh.at[src_v.at[j]], add=True)
        return carry

    lax.fori_loop(0, HCHUNKS, ch, 0)
    plsc.subcore_barrier()
    pltpu.sync_copy(acc_sh.at[pl.ds(s * 640, 640)],
                    out_hbm.at[c, pl.ds(s * 640, 640)])


SCH = 64
NCH = EP // 16 // SCH
GRP = 16
NGRP = NCH // GRP
KB = 4


@functools.partial(
    pl.kernel,
    out_type=jax.ShapeDtypeStruct((BATCH, NA, C), F32),
    mesh=_mesh,
    compiler_params=pltpu.CompilerParams(needs_layout_passes=False),
    scratch_types=[
        pltpu.VMEM((2, GRP, SCH), I32),
        pltpu.VMEM((2, GRP, SCH), I32),
        pltpu.VMEM((KB, SCH, C), F32),
        pltpu.VMEM_SHARED((NA, C), F32),
        pltpu.SemaphoreType.DMA,
        pltpu.SemaphoreType.DMA,
        pltpu.SemaphoreType.DMA,
        pltpu.SemaphoreType.DMA,
    ],
)
def _sc_asum(g_hbm, sabs_hbm, dst_hbm, out_hbm,
             sidx_v, didx_v, rows_v, acc_sh, sem_i, sem_g, sem_s, sem_z):
    c = lax.axis_index("c")
    s = lax.axis_index("s")
    zero16 = jnp.zeros((16,), F32)

    def wait_i():
        pltpu.make_async_copy(dst_hbm.at[pl.ds(0, GRP)], didx_v.at[0],
                              sem_i).wait()

    def wait_g():
        pltpu.make_async_copy(g_hbm.at[sidx_v.at[0, 0]], rows_v.at[0],
                              sem_g).wait()

    def wait_s():
        pltpu.make_async_copy(rows_v.at[0], acc_sh.at[didx_v.at[0, 0]],
                              sem_s).wait()

    for bi in range(4):
        b = c * 4 + bi
        for i in range(SCH):
            for u in range(8):
                rows_v[0, i, pl.ds(u * 16, 16)] = zero16
        for k in range(10):
            pltpu.async_copy(rows_v.at[0],
                             acc_sh.at[pl.ds(s * 640 + k * SCH, SCH)], sem_z)
        for k in range(10):
            pltpu.make_async_copy(rows_v.at[0],
                                  acc_sh.at[pl.ds(s * 640, SCH)],
                                  sem_z).wait()
        plsc.subcore_barrier()

        def issue_idx(g, ib):
            pltpu.async_copy(
                sabs_hbm.at[b, pl.ds(s * NCH + g * GRP, GRP)],
                sidx_v.at[ib], sem_i)
            pltpu.async_copy(
                dst_hbm.at[pl.ds(s * NCH + g * GRP, GRP)],
                didx_v.at[ib], sem_i)

        issue_idx(0, 0)
        issue_idx(1, 1)
        for g in range(NGRP):
            ib = g % 2
            wait_i()
            wait_i()
            pltpu.async_copy(g_hbm.at[sidx_v.at[ib, 0]], rows_v.at[0], sem_g)
            pltpu.async_copy(g_hbm.at[sidx_v.at[ib, 1]], rows_v.at[1], sem_g)

            def ck(k, carry):
                wait_g()
                pltpu.async_copy(rows_v.at[lax.rem(k, KB)],
                                 acc_sh.at[didx_v.at[ib, k]], sem_s, add=True)

                @pl.when(k >= 2)
                def _():
                    wait_s()

                @pl.when(k < GRP - 2)
                def _():
                    pltpu.async_copy(g_hbm.at[sidx_v.at[ib, k + 2]],
                                     rows_v.at[lax.rem(k + 2, KB)], sem_g)

                return carry

            lax.fori_loop(0, GRP, ck, 0)
            wait_s()
            wait_s()
            if g + 2 < NGRP:
                issue_idx(g + 2, ib)
        plsc.subcore_barrier()
        pltpu.sync_copy(acc_sh.at[pl.ds(s * 640, 640)],
                        out_hbm.at[b, pl.ds(s * 640, 640)])


def _tc_scale_body(x_r, p_r, xs_r, dis_r):
    @pl.when(pl.program_id(0) == 0)
    def _():
        deg = p_r[0, :, 0:1] + p_r[1, :, 0:1]
        dis_r[...] = jnp.where(
            deg > 0, 1.0 / jnp.sqrt(jnp.maximum(deg, 1.0)), 0.0)
    dis = dis_r[...]
    xs_r[0, :N, :] = x_r[0] * dis[:N]
    xs_r[0, N:, :] = jnp.zeros((NP - N, C), F32)


def _tc_g2_body(s_r, dis_r, g_r):
    dis = dis_r[...]
    d2 = dis * dis
    g_r[0, :N, :] = s_r[0, :N, :] * d2[:N]
    g_r[0, N:, :] = jnp.zeros((NP - N, C), F32)


def _tc_cheb_body(x_r, s1_r, s2_r, dis_r, wa_r, wb_r, wc_r, bias_r, r_r):
    dis = dis_r[...]
    d1 = s1_r[0] * dis
    d2 = s2_r[0] * dis
    h = (jnp.dot(x_r[0], wa_r[...], preferred_element_type=F32)
         + jnp.dot(d1, wb_r[...], preferred_element_type=F32)
         + jnp.dot(d2, wc_r[...], preferred_element_type=F32)
         + bias_r[...])
    r_r[0] = jnp.maximum(h, 0.0)


def _tc_bn_body(r_r, g_r, b_r, o_r):
    acc = r_r[0]
    for b in range(1, BATCH):
        acc = acc + r_r[b]
    mean = acc * (1.0 / BATCH)
    vacc = (r_r[0] - mean) * (r_r[0] - mean)
    for b in range(1, BATCH):
        d = r_r[b] - mean
        vacc = vacc + d * d
    inv = 1.0 / jnp.sqrt(vacc * (1.0 / BATCH) + 1e-5)
    gam = g_r[...]
    bet = b_r[...]
    for b in range(BATCH):
        y = (r_r[b] - mean) * inv
        o_r[b] = y[:, None, :] * gam + bet


NB = 1000


def kernel(x, adj_in, adj_out, W, bias, gamma, beta):
    src = adj_in[0]
    dst = adj_in[1]
    srcp = jnp.concatenate([src, jnp.full((PAD_E,), N, I32)])
    dstp = jnp.concatenate([dst, jnp.zeros((PAD_E,), I32)])
    srch = srcp.reshape(EP // CHUNK, CHUNK)
    sabs = (srcp[None, :]
            + (jnp.arange(BATCH, dtype=I32) * NP)[:, None]
            ).reshape(BATCH, EP // SCH, SCH)
    dsts = dstp.reshape(EP // SCH, SCH)

    partials = _sc_degree(srch)

    xs, dis = pl.pallas_call(
        _tc_scale_body,
        grid=(BATCH,),
        in_specs=[
            pl.BlockSpec((1, N, C), lambda j: (j, 0, 0)),
            pl.BlockSpec((2, NA, C), lambda j: (0, 0, 0)),
        ],
        out_specs=[
            pl.BlockSpec((1, NP, C), lambda j: (j, 0, 0)),
            pl.BlockSpec((NA, 1), lambda j: (0, 0)),
        ],
        out_shape=[
            jax.ShapeDtypeStruct((BATCH, NP, C), F32),
            jax.ShapeDtypeStruct((NA, 1), F32),
        ],
    )(x, partials)

    S1 = _sc_asum(xs.reshape(GROWS, C), sabs, dsts)

    g2 = pl.pallas_call(
        _tc_g2_body,
        grid=(BATCH,),
        in_specs=[
            pl.BlockSpec((1, NA, C), lambda j: (j, 0, 0)),
            pl.BlockSpec((NA, 1), lambda j: (0, 0)),
        ],
        out_specs=pl.BlockSpec((1, NP, C), lambda j: (j, 0, 0)),
        out_shape=jax.ShapeDtypeStruct((BATCH, NP, C), F32),
    )(S1, dis)

    S2 = _sc_asum(g2.reshape(GROWS, C), sabs, dsts)

    Wa = W[0] - W[2]
    Wb = -W[1]
    Wc = 2.0 * W[2]
    r = pl.pallas_call(
        _tc_cheb_body,
        grid=(N // NB, BATCH),
        in_specs=[
            pl.BlockSpec((1, NB, C), lambda i, j: (j, i, 0)),
            pl.BlockSpec((1, NB, C), lambda i, j: (j, i, 0)),
            pl.BlockSpec((1, NB, C), lambda i, j: (j, i, 0)),
            pl.BlockSpec((NB, 1), lambda i, j: (i, 0)),
            pl.BlockSpec((C, C), lambda i, j: (0, 0)),
            pl.BlockSpec((C, C), lambda i, j: (0, 0)),
            pl.BlockSpec((C, C), lambda i, j: (0, 0)),
            pl.BlockSpec((1, C), lambda i, j: (0, 0)),
        ],
        out_specs=pl.BlockSpec((1, NB, C), lambda i, j: (j, i, 0)),
        out_shape=jax.ShapeDtypeStruct((BATCH, N, C), F32),
    )(x, S1, S2, dis, Wa, Wb, Wc, bias.reshape(1, C))

    out4 = pl.pallas_call(
        _tc_bn_body,
        grid=(N // NB,),
        in_specs=[
            pl.BlockSpec((BATCH, NB, C), lambda i: (0, i, 0)),
            pl.BlockSpec((NB, 4, C), lambda i: (i, 0, 0)),
            pl.BlockSpec((NB, 4, C), lambda i: (i, 0, 0)),
        ],
        out_specs=pl.BlockSpec((BATCH, NB, 4, C), lambda i: (0, i, 0, 0)),
        out_shape=jax.ShapeDtypeStruct((BATCH, N, 4, C), F32),
    )(r, gamma.reshape(N, 4, C), beta.reshape(N, 4, C))

    return out4.reshape(BATCH, N * 4, C)


# speedup vs baseline: 1.1036x; 1.1036x over previous
import functools

import jax
import jax.numpy as jnp
from jax import lax
from jax.experimental import pallas as pl
from jax.experimental.pallas import tpu as pltpu
from jax.experimental.pallas import tpu_sc as plsc

F32 = jnp.float32
I32 = jnp.int32

N = 10000
NP = 10016
NA = 10112
NAD = 10240
BATCH = 8
C = 128
E = 160000
EP = 163840
PAD_E = EP - E
CHUNK = 128
CHUNKS_PER_TILE = EP // 16 // CHUNK
HIST_PER_TILE = EP // 32
GROWS = BATCH * NP

_mesh = plsc.VectorSubcoreMesh(
    core_axis_name="c", subcore_axis_name="s", num_cores=2, num_subcores=16)


HCHUNKS = EP // CHUNK // 32


@functools.partial(
    pl.kernel,
    out_type=jax.ShapeDtypeStruct((2, NAD, C), F32),
    mesh=_mesh,
    compiler_params=pltpu.CompilerParams(needs_layout_passes=False),
    scratch_types=[
        pltpu.VMEM((HCHUNKS, CHUNK), I32),
        pltpu.VMEM((CHUNK, C), F32),
        pltpu.VMEM((16, C), F32),
        pltpu.VMEM_SHARED((NA, C), F32),
    ],
)
def _sc_degree(src_hbm, out_hbm, src_v, ones_v, zero_v, acc_sh):
    c = lax.axis_index("c")
    s = lax.axis_index("s")
    wid = c * 16 + s
    pltpu.sync_copy(src_hbm.at[pl.ds(wid * HCHUNKS, HCHUNKS)], src_v)
    one16 = jnp.ones((16,), F32)
    zero16 = jnp.zeros((16,), F32)
    for i in range(CHUNK):
        for u in range(8):
            ones_v[i, pl.ds(u * 16, 16)] = one16
    for i in range(16):
        for u in range(8):
            zero_v[i, pl.ds(u * 16, 16)] = zero16
    for k in range(40):
        pltpu.sync_copy(zero_v, acc_sh.at[pl.ds(s * 640 + k * 16, 16)])
    plsc.subcore_barrier()

    def ch(j, carry):
        pltpu.sync_copy(ones_v, acc_sh.at[src_v.at[j]], add=True)
        return carry

    lax.fori_loop(0, HCHUNKS, ch, 0)
    plsc.subcore_barrier()
    pltpu.sync_copy(acc_sh.at[pl.ds(s * 640, 640)],
                    out_hbm.at[c, pl.ds(s * 640, 640)])


NCH = EP // 16 // CHUNK


@functools.partial(
    pl.kernel,
    out_type=jax.ShapeDtypeStruct((BATCH, NA, C), F32),
    mesh=_mesh,
    compiler_params=pltpu.CompilerParams(needs_layout_passes=False),
    scratch_types=[
        pltpu.VMEM((3, CHUNK), I32),
        pltpu.VMEM((4, CHUNK), I32),
        pltpu.VMEM((3, CHUNK, C), F32),
        pltpu.VMEM_SHARED((NA, C), F32),
        pltpu.SemaphoreType.DMA,
        pltpu.SemaphoreType.DMA,
        pltpu.SemaphoreType.DMA,
        pltpu.SemaphoreType.DMA,
    ],
)
def _sc_asum(g_hbm, sabs_hbm, dst_hbm, out_hbm,
             sidx_v, didx_v, rows_v, acc_sh, sem_i, sem_g, sem_s, sem_z):
    c = lax.axis_index("c")
    s = lax.axis_index("s")
    zero16 = jnp.zeros((16,), F32)

    def wait_i():
        pltpu.make_async_copy(dst_hbm.at[0], didx_v.at[0], sem_i).wait()

    def wait_g():
        pltpu.make_async_copy(g_hbm.at[sidx_v.at[0]], rows_v.at[0],
                              sem_g).wait()

    def wait_s():
        pltpu.make_async_copy(rows_v.at[0], acc_sh.at[didx_v.at[0]],
                              sem_s).wait()

    for bi in range(4):
        b = c * 4 + bi
        for i in range(CHUNK):
            for u in range(8):
                rows_v[0, i, pl.ds(u * 16, 16)] = zero16
        for q in range(4):
            pltpu.async_copy(rows_v.at[0],
                             acc_sh.at[pl.ds(s * 632 + q * CHUNK, CHUNK)],
                             sem_z)
        pltpu.async_copy(rows_v.at[0, pl.ds(0, 120)],
                         acc_sh.at[pl.ds(s * 632 + 512, 120)], sem_z)
        for q in range(4):
            pltpu.make_async_copy(rows_v.at[0],
                                  acc_sh.at[pl.ds(s * 632, CHUNK)],
                                  sem_z).wait()
        pltpu.make_async_copy(rows_v.at[0, pl.ds(0, 120)],
                              acc_sh.at[pl.ds(s * 632, 120)], sem_z).wait()
        plsc.subcore_barrier()

        def issue_idx(j):
            pltpu.async_copy(sabs_hbm.at[b, s * NCH + j],
                             sidx_v.at[lax.rem(j, 3)], sem_i)
            pltpu.async_copy(dst_hbm.at[s * NCH + j],
                             didx_v.at[lax.rem(j, 4)], sem_i)

        def fire(j):
            pltpu.async_copy(g_hbm.at[sidx_v.at[lax.rem(j, 3)]],
                             rows_v.at[lax.rem(j, 3)], sem_g)

        issue_idx(0)
        issue_idx(1)
        issue_idx(2)
        wait_i()
        wait_i()
        fire(0)
        wait_i()
        wait_i()
        fire(1)

        def ck(k, carry):
            wait_g()
            pltpu.async_copy(rows_v.at[lax.rem(k, 3)],
                             acc_sh.at[didx_v.at[lax.rem(k, 4)]], sem_s,
                             add=True)

            @pl.when(k >= 1)
            def _():
                wait_s()

            @pl.when(k < NCH - 2)
            def _():
                wait_i()
                wait_i()
                fire(k + 2)

            @pl.when(k < NCH - 3)
            def _():
                issue_idx(k + 3)

            return carry

        lax.fori_loop(0, NCH, ck, 0)
        wait_s()
        plsc.subcore_barrier()
        pltpu.sync_copy(acc_sh.at[pl.ds(s * 632, 632)],
                        out_hbm.at[b, pl.ds(s * 632, 632)])


def _tc_scale_body(x_r, p_r, xs_r, dis_r):
    @pl.when(pl.program_id(0) == 0)
    def _():
        deg = p_r[0, :NA, 0:1] + p_r[1, :NA, 0:1]
        dis_r[...] = jnp.where(
            deg > 0, 1.0 / jnp.sqrt(jnp.maximum(deg, 1.0)), 0.0)
    dis = dis_r[...]
    xs_r[0, :N, :] = x_r[0] * dis[:N]
    xs_r[0, N:, :] = jnp.zeros((NP - N, C), F32)


def _tc_g2_body(s_r, dis_r, g_r):
    dis = dis_r[...]
    d2 = dis * dis
    g_r[0, :N, :] = s_r[0, :N, :] * d2[:N]
    g_r[0, N:, :] = jnp.zeros((NP - N, C), F32)


def _tc_cheb_body(x_r, s1_r, s2_r, dis_r, wa_r, wb_r, wc_r, bias_r, r_r):
    dis = dis_r[...]
    d1 = s1_r[0] * dis
    d2 = s2_r[0] * dis
    h = (jnp.dot(x_r[0], wa_r[...], preferred_element_type=F32)
         + jnp.dot(d1, wb_r[...], preferred_element_type=F32)
         + jnp.dot(d2, wc_r[...], preferred_element_type=F32)
         + bias_r[...])
    r_r[0] = jnp.maximum(h, 0.0)


def _tc_bn_body(r_r, g_r, b_r, o_r):
    acc = r_r[0]
    for b in range(1, BATCH):
        acc = acc + r_r[b]
    mean = acc * (1.0 / BATCH)
    vacc = (r_r[0] - mean) * (r_r[0] - mean)
    for b in range(1, BATCH):
        d = r_r[b] - mean
        vacc = vacc + d * d
    inv = 1.0 / jnp.sqrt(vacc * (1.0 / BATCH) + 1e-5)
    gam = g_r[...]
    bet = b_r[...]
    for b in range(BATCH):
        y = (r_r[b] - mean) * inv
        o_r[b] = y[:, None, :] * gam + bet


NB = 1000


def kernel(x, adj_in, adj_out, W, bias, gamma, beta):
    src = adj_in[0]
    dst = adj_in[1]
    srcp = jnp.concatenate([src, jnp.full((PAD_E,), N, I32)])
    dstp = jnp.concatenate([dst, jnp.zeros((PAD_E,), I32)])
    srch = srcp.reshape(EP // CHUNK, CHUNK)
    sabs = (srcp[None, :]
            + (jnp.arange(BATCH, dtype=I32) * NP)[:, None]
            ).reshape(BATCH, EP // CHUNK, CHUNK)
    dsts = dstp.reshape(EP // CHUNK, CHUNK)

    partials = _sc_degree(srch)

    xs, dis = pl.pallas_call(
        _tc_scale_body,
        grid=(BATCH,),
        in_specs=[
            pl.BlockSpec((1, N, C), lambda j: (j, 0, 0)),
            pl.BlockSpec((2, NAD, C), lambda j: (0, 0, 0)),
        ],
        out_specs=[
            pl.BlockSpec((1, NP, C), lambda j: (j, 0, 0)),
            pl.BlockSpec((NA, 1), lambda j: (0, 0)),
        ],
        out_shape=[
            jax.ShapeDtypeStruct((BATCH, NP, C), F32),
            jax.ShapeDtypeStruct((NA, 1), F32),
        ],
    )(x, partials)

    S1 = _sc_asum(xs.reshape(GROWS, C), sabs, dsts)

    g2 = pl.pallas_call(
        _tc_g2_body,
        grid=(BATCH,),
        in_specs=[
            pl.BlockSpec((1, NA, C), lambda j: (j, 0, 0)),
            pl.BlockSpec((NA, 1), lambda j: (0, 0)),
        ],
        out_specs=pl.BlockSpec((1, NP, C), lambda j: (j, 0, 0)),
        out_shape=jax.ShapeDtypeStruct((BATCH, NP, C), F32),
    )(S1, dis)

    S2 = _sc_asum(g2.reshape(GROWS, C), sabs, dsts)

    Wa = W[0] - W[2]
    Wb = -W[1]
    Wc = 2.0 * W[2]
    r = pl.pallas_call(
        _tc_cheb_body,
        grid=(N // NB, BATCH),
        in_specs=[
            pl.BlockSpec((1, NB, C), lambda i, j: (j, i, 0)),
            pl.BlockSpec((1, NB, C), lambda i, j: (j, i, 0)),
            pl.BlockSpec((1, NB, C), lambda i, j: (j, i, 0)),
            pl.BlockSpec((NB, 1), lambda i, j: (i, 0)),
            pl.BlockSpec((C, C), lambda i, j: (0, 0)),
            pl.BlockSpec((C, C), lambda i, j: (0, 0)),
            pl.BlockSpec((C, C), lambda i, j: (0, 0)),
            pl.BlockSpec((1, C), lambda i, j: (0, 0)),
        ],
        out_specs=pl.BlockSpec((1, NB, C), lambda i, j: (j, i, 0)),
        out_shape=jax.ShapeDtypeStruct((BATCH, N, C), F32),
    )(x, S1, S2, dis, Wa, Wb, Wc, bias.reshape(1, C))

    out4 = pl.pallas_call(
        _tc_bn_body,
        grid=(N // NB,),
        in_specs=[
            pl.BlockSpec((BATCH, NB, C), lambda i: (0, i, 0)),
            pl.BlockSpec((NB, 4, C), lambda i: (i, 0, 0)),
            pl.BlockSpec((NB, 4, C), lambda i: (i, 0, 0)),
        ],
        out_specs=pl.BlockSpec((BATCH, NB, 4, C), lambda i: (0, i, 0, 0)),
        out_shape=jax.ShapeDtypeStruct((BATCH, N, 4, C), F32),
    )(r, gamma.reshape(N, 4, C), beta.reshape(N, 4, C))

    return out4.reshape(BATCH, N * 4, C)
